# Initial kernel scaffold; baseline (speedup 1.0000x reference)
import jax
import jax.numpy as jnp
from jax.experimental import pallas as pl


def kernel(x):
    raise NotImplementedError("write your pallas kernel here")


# speedup vs baseline: 13.2611x; 13.2611x over previous
import functools

import jax
import jax.numpy as jnp
from jax import lax
from jax.experimental import pallas as pl
from jax.experimental.pallas import tpu as pltpu
from jax.experimental.pallas import tpu_sc as plsc

K = 16
C = 96
NV = C // 16
NC, NS = 2, 16
NW = NC * NS

LOCS = 512 * 512
PER_W = LOCS // NW
CHUNK = 128
NCHUNKS = PER_W // CHUNK


def _mask_one_loc(j, xbuf, ybuf):
    v = [xbuf[j, pl.ds(16 * i, 16)] for i in range(NV)]

    def _sort(a):
        return plsc.sort_key_val(a, a)[0]

    t16 = _sort(v[0])
    for i in range(1, NV):
        s = _sort(v[i])
        t16 = _sort(jnp.maximum(t16, lax.rev(s, (0,))))
    t = jnp.min(t16)
    gt = [vi > t for vi in v]
    n_gt = jnp.int32(0)
    for g in gt:
        n_gt = n_gt + jnp.sum(g.astype(jnp.int32))
    need = K - n_gt
    base = jnp.int32(0)
    for i in range(NV):
        eq = v[i] == t
        eqi = eq.astype(jnp.int32)
        pref = base + plsc.cumsum(eqi) - eqi
        sel = gt[i] | (eq & (pref < need))
        ybuf[j, pl.ds(16 * i, 16)] = jnp.where(sel, 1.0, 0.0).astype(jnp.float32)
        base = base + jnp.sum(eqi)


@functools.partial(
    pl.kernel,
    out_type=jax.ShapeDtypeStruct((LOCS, C), jnp.float32),
    mesh=plsc.VectorSubcoreMesh(core_axis_name="c", subcore_axis_name="s"),
    scratch_types=[
        pltpu.VMEM((CHUNK, C), jnp.float32),
        pltpu.VMEM((CHUNK, C), jnp.float32),
    ],
    compiler_params=pltpu.CompilerParams(needs_layout_passes=False),
)
def _topk_mask_sc(x_hbm, y_hbm, xbuf, ybuf):
    wid = lax.axis_index("s") * NC + lax.axis_index("c")
    base = wid * PER_W

    def chunk_body(ci, carry):
        off = base + ci * CHUNK
        pltpu.sync_copy(x_hbm.at[pl.ds(off, CHUNK)], xbuf)

        def loc_body(j, c2):
            _mask_one_loc(j, xbuf, ybuf)
            return c2

        lax.fori_loop(0, CHUNK, loc_body, 0, unroll=2)
        pltpu.sync_copy(ybuf, y_hbm.at[pl.ds(off, CHUNK)])
        return carry

    lax.fori_loop(0, NCHUNKS, chunk_body, 0)


def kernel(x):
    B, H, W, Cx = x.shape
    assert (B, H * W, Cx) == (1, LOCS, C)
    y = _topk_mask_sc(x.reshape(LOCS, C))
    return y.reshape(B, H, W, Cx)


# speedup vs baseline: 25.3119x; 1.9087x over previous
import functools

import jax
import jax.numpy as jnp
from jax import lax
from jax.experimental import pallas as pl
from jax.experimental.pallas import tpu as pltpu
from jax.experimental.pallas import tpu_sc as plsc

K = 16
C = 96
NV = C // 16
NC, NS = 2, 16
NW = NC * NS

LOCS = 512 * 512
PER_W = LOCS // NW
CHUNK = 256
NCHUNKS = PER_W // CHUNK
NBUF = 2
G = 2


def _sort_asc(a):
    return plsc.sort_key_val(a, a)[0]


def _sort_desc(a):
    return plsc.sort_key_val(a, a, descending=True)[0]


def _exact_fixup(j, t, xbuf, ybuf):
    v = [xbuf[j, pl.ds(16 * i, 16)] for i in range(NV)]
    gt = [vi > t for vi in v]
    n_gt = jnp.zeros((16,), jnp.int32)
    for g in gt:
        n_gt = n_gt + plsc.all_reduce_population_count(g)
    need = jnp.full((16,), K, jnp.int32) - n_gt
    base = jnp.zeros((16,), jnp.int32)
    for i in range(NV):
        eq = v[i] == t
        eqi = eq.astype(jnp.int32)
        pref = base + plsc.cumsum(eqi) - eqi
        sel = gt[i] | (eq & (pref < need))
        ybuf[j, pl.ds(16 * i, 16)] = jnp.where(sel, 1.0, 0.0).astype(
            jnp.float32)
        base = base + plsc.all_reduce_population_count(eq)


def _masks_group(j0, xbuf, ybuf):
    V = [[xbuf[j0 + g, pl.ds(16 * i, 16)] for i in range(NV)]
         for g in range(G)]
    S = [[None] * NV for _ in range(G)]
    for i in range(NV):
        for g in range(G):
            S[g][i] = _sort_asc(V[g][i]) if i % 2 == 0 else _sort_desc(V[g][i])
    m01 = [_sort_asc(jnp.maximum(S[g][0], S[g][1])) for g in range(G)]
    m23 = [_sort_desc(jnp.maximum(S[g][2], S[g][3])) for g in range(G)]
    m45 = [_sort_desc(jnp.maximum(S[g][4], S[g][5])) for g in range(G)]
    m03 = [_sort_asc(jnp.maximum(m01[g], m23[g])) for g in range(G)]
    t16 = [_sort_asc(jnp.maximum(m03[g], m45[g])) for g in range(G)]
    ts = [t16[g][0] for g in range(G)]
    one, zero = jnp.int32(1), jnp.int32(0)
    acc = jnp.zeros((16,), jnp.int32)
    for g in range(G):
        for i in range(NV):
            ge = xbuf[j0 + g, pl.ds(16 * i, 16)] >= ts[g]
            acc = acc + jnp.where(ge, one, zero)
            ybuf[j0 + g, pl.ds(16 * i, 16)] = jnp.where(ge, 1.0, 0.0).astype(
                jnp.float32)
    total = plsc.cumsum(acc)[15]

    @pl.when(total != K * G)
    def _():
        for g in range(G):
            _exact_fixup(j0 + g, ts[g], xbuf, ybuf)


@functools.partial(
    pl.kernel,
    out_type=jax.ShapeDtypeStruct((LOCS, C), jnp.float32),
    mesh=plsc.VectorSubcoreMesh(core_axis_name="c", subcore_axis_name="s"),
    scratch_types=[
        pltpu.VMEM((NBUF, CHUNK, C), jnp.float32),
        pltpu.VMEM((NBUF, CHUNK, C), jnp.float32),
        pltpu.SemaphoreType.DMA((NBUF,)),
        pltpu.SemaphoreType.DMA((NBUF,)),
    ],
    compiler_params=pltpu.CompilerParams(needs_layout_passes=False),
)
def _topk_mask_sc(x_hbm, y_hbm, xbuf, ybuf, insem, outsem):
    wid = lax.axis_index("s") * NC + lax.axis_index("c")
    base = wid * PER_W

    def in_copy(ci, b):
        return pltpu.make_async_copy(
            x_hbm.at[pl.ds(base + ci * CHUNK, CHUNK)], xbuf.at[b],
            insem.at[b])

    def out_copy(ci, b):
        return pltpu.make_async_copy(
            ybuf.at[b], y_hbm.at[pl.ds(base + ci * CHUNK, CHUNK)],
            outsem.at[b])

    in_copy(0, 0).start()

    def round_body(r, carry):
        for b in range(NBUF):
            ci = NBUF * r + b
            nxt = ci + 1

            @pl.when(nxt < NCHUNKS)
            def _():
                in_copy(nxt, (b + 1) % NBUF).start()

            in_copy(ci, b).wait()

            @pl.when(ci >= NBUF)
            def _():
                out_copy(ci - NBUF, b).wait()

            @plsc.parallel_loop(0, CHUNK, step=G)
            def _loc(j0):
                _masks_group(j0, xbuf.at[b], ybuf.at[b])

            out_copy(ci, b).start()
        return carry

    lax.fori_loop(0, NCHUNKS // NBUF, round_body, 0)
    out_copy(NCHUNKS - NBUF, 0).wait()
    out_copy(NCHUNKS - 1, 1).wait()


def kernel(x):
    B, H, W, Cx = x.shape
    assert (B, H * W, Cx) == (1, LOCS, C)
    y = _topk_mask_sc(x.reshape(LOCS, C))
    return y.reshape(B, H, W, Cx)
